# Initial kernel scaffold; baseline (speedup 1.0000x reference)
import jax
import jax.numpy as jnp
from jax.experimental import pallas as pl


def kernel(z, W_in, b_in, W_out, b_out, codebook):
    raise NotImplementedError("write your pallas kernel here")


# speedup vs baseline: 1.5238x; 1.5238x over previous
import functools

import jax
import jax.numpy as jnp
from jax import lax
from jax.experimental import pallas as pl
from jax.experimental.pallas import tpu as pltpu
import jax.experimental.pallas.tpu_sc as plsc

T_TILE = 1024
K_CHUNK = 2048


def _encode_body(z_ref, w_in_ref, b_in_ref, cbt_ref, idx_ref):
    ze = lax.dot_general(w_in_ref[...], z_ref[...], (((1,), (0,)), ((), ())),
                         preferred_element_type=jnp.float32)
    ze = ze + b_in_ref[...]
    s2 = jnp.sum(ze * ze, axis=0, keepdims=True)
    n = jnp.maximum(jnp.sqrt(s2), 1e-12)
    en = ze / n
    t1 = jnp.sum(en * en, axis=0, keepdims=True).T

    cbt = cbt_ref[...]
    m2 = jnp.sum(cbt * cbt, axis=0, keepdims=True)
    m = jnp.maximum(jnp.sqrt(m2), 1e-12)
    cn = cbt / m
    t3 = jnp.sum(cn * cn, axis=0, keepdims=True)

    k = cbt.shape[1]
    rev = lax.broadcasted_iota(jnp.int32, (T_TILE, K_CHUNK), 1)
    rev = K_CHUNK - rev
    best = jnp.full((T_TILE,), jnp.inf, jnp.float32)
    bidx = jnp.zeros((T_TILE,), jnp.int32)
    for c in range(k // K_CHUNK):
        cn_c = cn[:, c * K_CHUNK:(c + 1) * K_CHUNK]
        t3_c = t3[:, c * K_CHUNK:(c + 1) * K_CHUNK]
        sim = lax.dot_general(en, cn_c, (((0,), (0,)), ((), ())),
                              preferred_element_type=jnp.float32)
        dist = (t1 - 2.0 * sim) + t3_c
        mn = jnp.min(dist, axis=1)
        hit = dist <= mn[:, None]
        a = K_CHUNK - jnp.max(jnp.where(hit, rev, 0), axis=1)
        a = a + c * K_CHUNK
        take = mn < best
        best = jnp.where(take, mn, best)
        bidx = jnp.where(take, a, bidx)
    idx_ref[...] = bidx[None, None, :]


def _encode(z, w_in, b_in, cbt):
    t = z.shape[1]
    n_t = t // T_TILE
    idx = pl.pallas_call(
        _encode_body,
        grid=(n_t,),
        in_specs=[
            pl.BlockSpec((z.shape[0], T_TILE), lambda i: (0, i)),
            pl.BlockSpec(w_in.shape, lambda i: (0, 0)),
            pl.BlockSpec((w_in.shape[0], 1), lambda i: (0, 0)),
            pl.BlockSpec(cbt.shape, lambda i: (0, 0)),
        ],
        out_specs=pl.BlockSpec((1, 1, T_TILE), lambda i: (i, 0, 0)),
        out_shape=jax.ShapeDtypeStruct((n_t, 1, T_TILE), jnp.int32),
    )(z, w_in, b_in, cbt)
    return idx.reshape(t)


def _make_gather(v, d, b):
    info = plsc.get_sparse_core_info()
    nw = info.num_cores * info.num_subcores
    b_per_w = b // nw
    mesh = plsc.VectorSubcoreMesh(core_axis_name="c", subcore_axis_name="s")

    @functools.partial(
        pl.kernel, mesh=mesh,
        compiler_params=pltpu.CompilerParams(use_tc_tiling_on_sc=False),
        out_type=jax.ShapeDtypeStruct((b, d), jnp.float32),
        scratch_types=[
            pltpu.VMEM((b_per_w,), jnp.int32),
            pltpu.VMEM((b_per_w, d), jnp.float32),
            pltpu.SemaphoreType.DMA,
        ],
    )
    def gather(table_hbm, idx_hbm, out_hbm, idx_v, rows_v, sem):
        wid = lax.axis_index("s") * info.num_cores + lax.axis_index("c")
        base = wid * b_per_w
        pltpu.sync_copy(idx_hbm.at[pl.ds(base, b_per_w)], idx_v)
        pltpu.async_copy(table_hbm.at[idx_v], rows_v, sem).wait()
        pltpu.sync_copy(rows_v, out_hbm.at[pl.ds(base, b_per_w)])

    return gather


def _decode_body(rows_ref, w_out_ref, b_out_ref, out_ref):
    out = lax.dot_general(w_out_ref[...], rows_ref[...],
                          (((1,), (1,)), ((), ())),
                          preferred_element_type=jnp.float32)
    out_ref[...] = out + b_out_ref[...]


def _decode(rows, w_out, b_out):
    t = rows.shape[0]
    n_t = t // T_TILE
    return pl.pallas_call(
        _decode_body,
        grid=(n_t,),
        in_specs=[
            pl.BlockSpec((T_TILE, rows.shape[1]), lambda i: (i, 0)),
            pl.BlockSpec(w_out.shape, lambda i: (0, 0)),
            pl.BlockSpec((w_out.shape[0], 1), lambda i: (0, 0)),
        ],
        out_specs=pl.BlockSpec((w_out.shape[0], T_TILE), lambda i: (0, i)),
        out_shape=jax.ShapeDtypeStruct((w_out.shape[0], t), jnp.float32),
    )(rows, w_out, b_out)


def kernel(z, W_in, b_in, W_out, b_out, codebook):
    t = z.shape[1]
    indices = _encode(z, W_in, b_in.reshape(-1, 1), codebook.T)
    rows = _make_gather(codebook.shape[0], codebook.shape[1], t)(
        codebook, indices)
    z_q = _decode(rows, W_out, b_out.reshape(-1, 1))
    return (z_q, indices)
